# baseline (speedup 1.0000x reference)
import jax
import jax.numpy as jnp
from jax import lax
from jax.experimental import pallas as pl
from jax.experimental.pallas import tpu as pltpu
from jax.experimental.pallas import tpu_sc as plsc

N_USERS = 1000000
N_MOVIES = 100000
N_FACTORS = 128
BATCH = 16384

NC = 2
NS = 16
L = 16
NW = NC * NS
PER_W = BATCH // NW
CHUNK = 128
NCHUNK = PER_W // CHUNK
NG = CHUNK // L

_COMPILER_PARAMS = pltpu.CompilerParams(
    needs_layout_passes=False, use_tc_tiling_on_sc=False)

UB_BLK = 8192
UB_NFULL = N_USERS // UB_BLK
UB_TAIL = N_USERS - UB_NFULL * UB_BLK
MB_BLK = 2048
MB_NFULL = N_MOVIES // MB_BLK
MB_TAIL = N_MOVIES - MB_NFULL * MB_BLK


def _repack(buf2d, buf1d, n):
    iota16 = lax.iota(jnp.int32, L)
    zeros16 = jnp.zeros((L,), jnp.int32)

    @plsc.parallel_loop(0, n, step=L, unroll=4)
    def _(i):
        items = iota16 + i
        buf1d[pl.ds(i, L)] = plsc.load_gather(buf2d, [items, zeros16])


def _relayout_body(ubt_hbm, mbt_hbm, ub_out, mb_out,
                   ubuf, ubuf1, mbuf, mbuf1, sem0, sem1):
    wid = lax.axis_index("s") * NC + lax.axis_index("c")

    for i in range((UB_NFULL + NW - 1) // NW):
        blk = wid + i * NW

        @pl.when(blk < UB_NFULL)
        def _():
            off = blk * UB_BLK
            pltpu.async_copy(
                ubt_hbm.at[pl.ds(off, UB_BLK)], ubuf, sem0).wait()
            _repack(ubuf, ubuf1, UB_BLK)
            pltpu.sync_copy(ubuf1, ub_out.at[pl.ds(off, UB_BLK)])

    for i in range((MB_NFULL + NW - 1) // NW):
        blk = wid + i * NW

        @pl.when(blk < MB_NFULL)
        def _():
            off = blk * MB_BLK
            pltpu.async_copy(
                mbt_hbm.at[pl.ds(off, MB_BLK)], mbuf, sem1).wait()
            _repack(mbuf, mbuf1, MB_BLK)
            pltpu.sync_copy(mbuf1, mb_out.at[pl.ds(off, MB_BLK)])

    @pl.when(wid == 0)
    def _():
        off = UB_NFULL * UB_BLK
        pltpu.async_copy(
            ubt_hbm.at[pl.ds(off, UB_TAIL)],
            ubuf.at[pl.ds(0, UB_TAIL)], sem0).wait()
        _repack(ubuf, ubuf1, UB_TAIL)
        pltpu.sync_copy(ubuf1.at[pl.ds(0, UB_TAIL)],
                        ub_out.at[pl.ds(off, UB_TAIL)])

    @pl.when(wid == 1)
    def _():
        off = MB_NFULL * MB_BLK
        pltpu.async_copy(
            mbt_hbm.at[pl.ds(off, MB_TAIL)],
            mbuf.at[pl.ds(0, MB_TAIL)], sem1).wait()
        _repack(mbuf, mbuf1, MB_TAIL)
        pltpu.sync_copy(mbuf1.at[pl.ds(0, MB_TAIL)],
                        mb_out.at[pl.ds(off, MB_TAIL)])


def _sc_body(users_hbm, movies_hbm, ut_hbm, mt_hbm, ubt_hbm, mbt_hbm,
             gb_hbm, out_hbm,
             uidx, midx, uq, mq, urows, mrows, ubrows, mbrows, gbv, out_v,
             sems):
    wid = lax.axis_index("s") * NC + lax.axis_index("c")
    base = wid * PER_W

    pltpu.sync_copy(gb_hbm, gbv.at[pl.ds(0, 1)])
    gb_vec = jnp.broadcast_to(gbv[:][0], (L,))

    zeros16 = jnp.zeros((L,), jnp.int32)
    iota16 = lax.iota(jnp.int32, L)
    items_g = [iota16 + (g * L) for g in range(NG)]

    def issue(c, s):
        cbase = base + c * CHUNK
        pltpu.sync_copy(users_hbm.at[pl.ds(cbase, CHUNK)], uidx.at[s])
        pltpu.sync_copy(movies_hbm.at[pl.ds(cbase, CHUNK)], midx.at[s])
        for g in range(NG):
            sl = pl.ds(g * L, L)
            uq[s, sl] = lax.shift_right_logical(uidx[s, sl], 4)
            mq[s, sl] = lax.shift_right_logical(midx[s, sl], 4)
        return (
            pltpu.async_copy(ut_hbm.at[uidx.at[s]], urows.at[s], sems.at[s, 0]),
            pltpu.async_copy(mt_hbm.at[midx.at[s]], mrows.at[s], sems.at[s, 1]),
            pltpu.async_copy(ubt_hbm.at[uq.at[s]], ubrows.at[s], sems.at[s, 2]),
            pltpu.async_copy(mbt_hbm.at[mq.at[s]], mbrows.at[s], sems.at[s, 3]),
        )

    descs = [None, None]
    descs[0] = issue(0, 0)
    for c in range(NCHUNK):
        s = c % 2
        if c + 1 < NCHUNK:
            descs[(c + 1) % 2] = issue(c + 1, (c + 1) % 2)
        for d in descs[s]:
            d.wait()

        ur = urows.at[s]
        mr = mrows.at[s]

        @plsc.parallel_loop(
            0, N_FACTORS, carry=tuple(
                jnp.zeros((L,), jnp.float32) for _ in range(NG)))
        def accs(f, accs_in):
            fv = zeros16 + f
            return tuple(
                acc + plsc.load_gather(ur, [items_g[g], fv])
                * plsc.load_gather(mr, [items_g[g], fv])
                for g, acc in enumerate(accs_in)
            )

        for g in range(NG):
            sl = pl.ds(g * L, L)
            ucol = jnp.bitwise_and(uidx[s, sl], 15)
            mcol = jnp.bitwise_and(midx[s, sl], 15)
            ubias = plsc.load_gather(ubrows.at[s], [items_g[g], ucol])
            mbias = plsc.load_gather(mbrows.at[s], [items_g[g], mcol])
            res = accs[g] + ubias + mbias + gb_vec
            out_v[pl.ds(c * CHUNK + g * L, L)] = res

    pltpu.sync_copy(out_v, out_hbm.at[pl.ds(base, PER_W)])


@jax.jit
def kernel(users, movies, user_table, movie_table, user_bias_table,
           movie_bias_table, global_bias):
    users = users.astype(jnp.int32)
    movies = movies.astype(jnp.int32)

    mesh = plsc.VectorSubcoreMesh(core_axis_name="c", subcore_axis_name="s")

    relayout = pl.kernel(
        _relayout_body,
        out_type=(jax.ShapeDtypeStruct((N_USERS,), jnp.float32),
                  jax.ShapeDtypeStruct((N_MOVIES,), jnp.float32)),
        mesh=mesh,
        compiler_params=_COMPILER_PARAMS,
        scratch_types=[
            pltpu.VMEM((UB_BLK, 1), jnp.float32),
            pltpu.VMEM((UB_BLK,), jnp.float32),
            pltpu.VMEM((MB_BLK, 1), jnp.float32),
            pltpu.VMEM((MB_BLK,), jnp.float32),
            pltpu.SemaphoreType.DMA,
            pltpu.SemaphoreType.DMA,
        ],
    )
    ub_flat, mb_flat = relayout(user_bias_table, movie_bias_table)
    ubt16 = ub_flat.reshape(N_USERS // L, L)
    mbt16 = mb_flat.reshape(N_MOVIES // L, L)

    run = pl.kernel(
        _sc_body,
        out_type=jax.ShapeDtypeStruct((BATCH,), jnp.float32),
        mesh=mesh,
        compiler_params=_COMPILER_PARAMS,
        scratch_types=[
            pltpu.VMEM((2, CHUNK), jnp.int32),
            pltpu.VMEM((2, CHUNK), jnp.int32),
            pltpu.VMEM((2, CHUNK), jnp.int32),
            pltpu.VMEM((2, CHUNK), jnp.int32),
            pltpu.VMEM((2, CHUNK, N_FACTORS), jnp.float32),
            pltpu.VMEM((2, CHUNK, N_FACTORS), jnp.float32),
            pltpu.VMEM((2, CHUNK, L), jnp.float32),
            pltpu.VMEM((2, CHUNK, L), jnp.float32),
            pltpu.VMEM((L,), jnp.float32),
            pltpu.VMEM((PER_W,), jnp.float32),
            pltpu.SemaphoreType.DMA((2, 4)),
        ],
    )
    return run(users, movies, user_table, movie_table, ubt16,
               mbt16, global_bias)


# speedup vs baseline: 10.1836x; 10.1836x over previous
import jax
import jax.numpy as jnp
from jax import lax
from jax.experimental import pallas as pl
from jax.experimental.pallas import tpu as pltpu
from jax.experimental.pallas import tpu_sc as plsc

N_USERS = 1000000
N_MOVIES = 100000
N_FACTORS = 128
BATCH = 16384

NC = 2
NS = 16
L = 16
NW = NC * NS
PER_W = BATCH // NW
CHUNK = 128
NCHUNK = PER_W // CHUNK
NG = CHUNK // L

_COMPILER_PARAMS = pltpu.CompilerParams(
    needs_layout_passes=False, use_tc_tiling_on_sc=False)


def _dot_body(users_hbm, movies_hbm, ut_hbm, mt_hbm, gb_hbm, out_hbm,
              uidx, midx, urows, mrows, gbv, out_v, sems):
    wid = lax.axis_index("s") * NC + lax.axis_index("c")
    base = wid * PER_W

    pltpu.sync_copy(gb_hbm, gbv.at[pl.ds(0, 1)])
    gb_vec = jnp.broadcast_to(gbv[:][0], (L,))

    zeros16 = jnp.zeros((L,), jnp.int32)
    iota16 = lax.iota(jnp.int32, L)
    items_g = [iota16 + (g * L) for g in range(NG)]

    def issue(c, s):
        cbase = base + c * CHUNK
        pltpu.sync_copy(users_hbm.at[pl.ds(cbase, CHUNK)], uidx.at[s])
        pltpu.sync_copy(movies_hbm.at[pl.ds(cbase, CHUNK)], midx.at[s])
        return (
            pltpu.async_copy(ut_hbm.at[uidx.at[s]], urows.at[s], sems.at[s, 0]),
            pltpu.async_copy(mt_hbm.at[midx.at[s]], mrows.at[s], sems.at[s, 1]),
        )

    descs = [None, None]
    descs[0] = issue(0, 0)
    for c in range(NCHUNK):
        s = c % 2
        if c + 1 < NCHUNK:
            descs[(c + 1) % 2] = issue(c + 1, (c + 1) % 2)
        for d in descs[s]:
            d.wait()

        ur = urows.at[s]
        mr = mrows.at[s]

        @plsc.parallel_loop(
            0, N_FACTORS, carry=tuple(
                jnp.zeros((L,), jnp.float32) for _ in range(NG)))
        def accs(f, accs_in):
            fv = zeros16 + f
            return tuple(
                acc + plsc.load_gather(ur, [items_g[g], fv])
                * plsc.load_gather(mr, [items_g[g], fv])
                for g, acc in enumerate(accs_in)
            )

        for g in range(NG):
            out_v[pl.ds(c * CHUNK + g * L, L)] = accs[g] + gb_vec

    pltpu.sync_copy(out_v, out_hbm.at[pl.ds(base, PER_W)])


def _bias_body(users_hbm, movies_hbm, ubt_hbm, mbt_hbm, partial_hbm, out_hbm,
               uidx, midx, uq, mq, ubrows, mbrows, part_v, out_v, sems):
    wid = lax.axis_index("s") * NC + lax.axis_index("c")
    base = wid * PER_W

    zeros16 = jnp.zeros((L,), jnp.int32)
    iota16 = lax.iota(jnp.int32, L)
    items_g = [iota16 + (g * L) for g in range(NG)]

    pltpu.sync_copy(users_hbm.at[pl.ds(base, PER_W)], uidx)
    pltpu.sync_copy(movies_hbm.at[pl.ds(base, PER_W)], midx)
    cp0 = pltpu.async_copy(partial_hbm.at[pl.ds(base, PER_W)], part_v,
                           sems.at[0])

    @plsc.parallel_loop(0, PER_W, step=L, unroll=4)
    def _(i):
        uq[pl.ds(i, L)] = lax.shift_right_logical(uidx[pl.ds(i, L)], 4)
        mq[pl.ds(i, L)] = lax.shift_right_logical(midx[pl.ds(i, L)], 4)

    cps = []
    for c in range(NCHUNK):
        csl = pl.ds(c * CHUNK, CHUNK)
        cps.append(pltpu.async_copy(
            ubt_hbm.at[uq.at[csl]], ubrows.at[csl], sems.at[1]))
        cps.append(pltpu.async_copy(
            mbt_hbm.at[mq.at[csl]], mbrows.at[csl], sems.at[2]))
    cp0.wait()
    for cp in cps:
        cp.wait()

    for c in range(NCHUNK):
        for g in range(NG):
            o = c * CHUNK + g * L
            sl = pl.ds(o, L)
            rows = iota16 + o
            ucol = jnp.bitwise_and(uidx[sl], 15)
            mcol = jnp.bitwise_and(midx[sl], 15)
            ubias = plsc.load_gather(ubrows, [rows, ucol])
            mbias = plsc.load_gather(mbrows, [rows, mcol])
            out_v[sl] = part_v[sl] + ubias + mbias

    pltpu.sync_copy(out_v, out_hbm.at[pl.ds(base, PER_W)])


@jax.jit
def kernel(users, movies, user_table, movie_table, user_bias_table,
           movie_bias_table, global_bias):
    users = users.astype(jnp.int32)
    movies = movies.astype(jnp.int32)
    ubt16 = user_bias_table.reshape(N_USERS // L, L)
    mbt16 = movie_bias_table.reshape(N_MOVIES // L, L)

    mesh = plsc.VectorSubcoreMesh(core_axis_name="c", subcore_axis_name="s")

    dot_run = pl.kernel(
        _dot_body,
        out_type=jax.ShapeDtypeStruct((BATCH,), jnp.float32),
        mesh=mesh,
        compiler_params=_COMPILER_PARAMS,
        scratch_types=[
            pltpu.VMEM((2, CHUNK), jnp.int32),
            pltpu.VMEM((2, CHUNK), jnp.int32),
            pltpu.VMEM((2, CHUNK, N_FACTORS), jnp.float32),
            pltpu.VMEM((2, CHUNK, N_FACTORS), jnp.float32),
            pltpu.VMEM((L,), jnp.float32),
            pltpu.VMEM((PER_W,), jnp.float32),
            pltpu.SemaphoreType.DMA((2, 2)),
        ],
    )
    partial = dot_run(users, movies, user_table, movie_table, global_bias)

    bias_run = pl.kernel(
        _bias_body,
        out_type=jax.ShapeDtypeStruct((BATCH,), jnp.float32),
        mesh=mesh,
        compiler_params=_COMPILER_PARAMS,
        scratch_types=[
            pltpu.VMEM((PER_W,), jnp.int32),
            pltpu.VMEM((PER_W,), jnp.int32),
            pltpu.VMEM((PER_W,), jnp.int32),
            pltpu.VMEM((PER_W,), jnp.int32),
            pltpu.VMEM((PER_W, L), jnp.float32),
            pltpu.VMEM((PER_W, L), jnp.float32),
            pltpu.VMEM((PER_W,), jnp.float32),
            pltpu.VMEM((PER_W,), jnp.float32),
            pltpu.SemaphoreType.DMA((3,)),
        ],
    )
    return bias_run(users, movies, ubt16, mbt16, partial)


# speedup vs baseline: 16.3492x; 1.6055x over previous
import jax
import jax.numpy as jnp
from jax import lax
from jax.experimental import pallas as pl
from jax.experimental.pallas import tpu as pltpu
from jax.experimental.pallas import tpu_sc as plsc

N_USERS = 1000000
N_MOVIES = 100000
N_FACTORS = 128
BATCH = 16384

NC = 2
NS = 16
L = 16
NW = NC * NS
PER_W = BATCH // NW
CHUNK = 128
NCHUNK = PER_W // CHUNK
NG = CHUNK // L

_COMPILER_PARAMS = pltpu.CompilerParams(
    needs_layout_passes=False, use_tc_tiling_on_sc=False)


def _dot_body(users_hbm, movies_hbm, ut_hbm, mt_hbm, gb_hbm, out_hbm,
              uidx, midx, urows, mrows, gbv, out_v, sems):
    wid = lax.axis_index("s") * NC + lax.axis_index("c")
    base = wid * PER_W

    pltpu.sync_copy(gb_hbm, gbv.at[pl.ds(0, 1)])
    gb_vec = jnp.broadcast_to(gbv[:][0], (L,))

    zeros16 = jnp.zeros((L,), jnp.int32)
    iota16 = lax.iota(jnp.int32, L)
    items_g = [iota16 + (g * L) for g in range(NG)]

    def issue(c, s):
        cbase = base + c * CHUNK
        pltpu.sync_copy(users_hbm.at[pl.ds(cbase, CHUNK)], uidx.at[s])
        pltpu.sync_copy(movies_hbm.at[pl.ds(cbase, CHUNK)], midx.at[s])
        return (
            pltpu.async_copy(ut_hbm.at[uidx.at[s]], urows.at[s], sems.at[s, 0]),
            pltpu.async_copy(mt_hbm.at[midx.at[s]], mrows.at[s], sems.at[s, 1]),
        )

    descs = [None, None]
    descs[0] = issue(0, 0)
    for c in range(NCHUNK):
        s = c % 2
        if c + 1 < NCHUNK:
            descs[(c + 1) % 2] = issue(c + 1, (c + 1) % 2)
        for d in descs[s]:
            d.wait()

        ur = urows.at[s]
        mr = mrows.at[s]

        @plsc.parallel_loop(
            0, N_FACTORS, carry=tuple(
                jnp.zeros((L,), jnp.float32) for _ in range(NG)))
        def accs(f, accs_in):
            fv = jnp.bitwise_and(iota16 + f, N_FACTORS - 1)
            return tuple(
                acc + plsc.load_gather(ur, [items_g[g], fv])
                * plsc.load_gather(mr, [items_g[g], fv])
                for g, acc in enumerate(accs_in)
            )

        for g in range(NG):
            out_v[pl.ds(c * CHUNK + g * L, L)] = accs[g] + gb_vec

    pltpu.sync_copy(out_v, out_hbm.at[pl.ds(base, PER_W)])


def _bias_body(users_hbm, movies_hbm, ubt_hbm, mbt_hbm, partial_hbm, out_hbm,
               uidx, midx, uq, mq, ubrows, mbrows, part_v, out_v, sems):
    wid = lax.axis_index("s") * NC + lax.axis_index("c")
    base = wid * PER_W

    zeros16 = jnp.zeros((L,), jnp.int32)
    iota16 = lax.iota(jnp.int32, L)
    items_g = [iota16 + (g * L) for g in range(NG)]

    pltpu.sync_copy(users_hbm.at[pl.ds(base, PER_W)], uidx)
    pltpu.sync_copy(movies_hbm.at[pl.ds(base, PER_W)], midx)
    cp0 = pltpu.async_copy(partial_hbm.at[pl.ds(base, PER_W)], part_v,
                           sems.at[0])

    @plsc.parallel_loop(0, PER_W, step=L, unroll=4)
    def _(i):
        uq[pl.ds(i, L)] = lax.shift_right_logical(uidx[pl.ds(i, L)], 4)
        mq[pl.ds(i, L)] = lax.shift_right_logical(midx[pl.ds(i, L)], 4)

    cps = []
    for c in range(NCHUNK):
        csl = pl.ds(c * CHUNK, CHUNK)
        cps.append(pltpu.async_copy(
            ubt_hbm.at[uq.at[csl]], ubrows.at[csl], sems.at[1]))
        cps.append(pltpu.async_copy(
            mbt_hbm.at[mq.at[csl]], mbrows.at[csl], sems.at[2]))
    cp0.wait()
    for cp in cps:
        cp.wait()

    for c in range(NCHUNK):
        for g in range(NG):
            o = c * CHUNK + g * L
            sl = pl.ds(o, L)
            rows = iota16 + o
            ucol = jnp.bitwise_and(uidx[sl], 15)
            mcol = jnp.bitwise_and(midx[sl], 15)
            ubias = plsc.load_gather(ubrows, [rows, ucol])
            mbias = plsc.load_gather(mbrows, [rows, mcol])
            out_v[sl] = part_v[sl] + ubias + mbias

    pltpu.sync_copy(out_v, out_hbm.at[pl.ds(base, PER_W)])


@jax.jit
def kernel(users, movies, user_table, movie_table, user_bias_table,
           movie_bias_table, global_bias):
    users = users.astype(jnp.int32)
    movies = movies.astype(jnp.int32)
    ubt16 = user_bias_table.reshape(N_USERS // L, L)
    mbt16 = movie_bias_table.reshape(N_MOVIES // L, L)

    mesh = plsc.VectorSubcoreMesh(core_axis_name="c", subcore_axis_name="s")

    dot_run = pl.kernel(
        _dot_body,
        out_type=jax.ShapeDtypeStruct((BATCH,), jnp.float32),
        mesh=mesh,
        compiler_params=_COMPILER_PARAMS,
        scratch_types=[
            pltpu.VMEM((2, CHUNK), jnp.int32),
            pltpu.VMEM((2, CHUNK), jnp.int32),
            pltpu.VMEM((2, CHUNK, N_FACTORS), jnp.float32),
            pltpu.VMEM((2, CHUNK, N_FACTORS), jnp.float32),
            pltpu.VMEM((L,), jnp.float32),
            pltpu.VMEM((PER_W,), jnp.float32),
            pltpu.SemaphoreType.DMA((2, 2)),
        ],
    )
    partial = dot_run(users, movies, user_table, movie_table, global_bias)

    bias_run = pl.kernel(
        _bias_body,
        out_type=jax.ShapeDtypeStruct((BATCH,), jnp.float32),
        mesh=mesh,
        compiler_params=_COMPILER_PARAMS,
        scratch_types=[
            pltpu.VMEM((PER_W,), jnp.int32),
            pltpu.VMEM((PER_W,), jnp.int32),
            pltpu.VMEM((PER_W,), jnp.int32),
            pltpu.VMEM((PER_W,), jnp.int32),
            pltpu.VMEM((PER_W, L), jnp.float32),
            pltpu.VMEM((PER_W, L), jnp.float32),
            pltpu.VMEM((PER_W,), jnp.float32),
            pltpu.VMEM((PER_W,), jnp.float32),
            pltpu.SemaphoreType.DMA((3,)),
        ],
    )
    return bias_run(users, movies, ubt16, mbt16, partial)
